# baseline (speedup 1.0000x reference)
import functools

import jax
import jax.numpy as jnp
from jax import lax
from jax.experimental import pallas as pl
from jax.experimental.pallas import tpu as pltpu
from jax.experimental.pallas import tpu_sc as plsc

NUM_EMB = 1000000
EMBED_DIM = 64
BATCH = 1024
SEQ_LEN = 200
N_TOK = BATCH * SEQ_LEN
N_WORKERS = 32

CB = 384
NBLK = NUM_EMB // CB
TAIL = NUM_EMB - NBLK * CB
NBLK_LO = NBLK // N_WORKERS
NBLK_REM = NBLK - NBLK_LO * N_WORKERS

PER_W = N_TOK // N_WORKERS
CHUNK = 800
N_CHUNK = PER_W // CHUNK
NBUF = 2


NUM_EMB_PAD = NBLK * CB + 128


def _sc_transpose(tab_hbm, fix_hbm, out_hbm, st0, st1, ob0, ob1,
                  ism0, ism1, osm0, osm1):
    wid = lax.axis_index("s") * 2 + lax.axis_index("c")
    nblk = NBLK_LO + jnp.where(wid < NBLK_REM, 1, 0)

    sts = [st0, st1]
    obs = [ob0, ob1]
    isms = [ism0, ism1]
    osms = [osm0, osm1]

    iota = lax.broadcasted_iota(jnp.int32, (16,), 0)
    rowv = [iota + 16 * m for m in range(4)]

    def transpose_block(st, ob, ncol):
        def rbody(r, carry):
            for m in range(8):
                col = jnp.full((16,), 2 * r + m // 4, jnp.int32)
                vals = plsc.load_gather(st, [rowv[m % 4], col])
                ob[r, pl.ds(16 * m, 16)] = vals
            return carry
        lax.fori_loop(0, ncol // 2, rbody, 0)

    def bbody(i, carry):
        for b in range(NBUF):
            t = i * NBUF + b

            @pl.when((t >= NBUF) & (t < nblk))
            def _drain():
                pltpu.make_async_copy(
                    obs[b], out_hbm.at[pl.ds(0, CB // 2)], osms[b]).wait()

            @pl.when(t < nblk)
            def _go():
                kb = wid + t * N_WORKERS
                c0 = pl.multiple_of(kb * CB, 128)
                pltpu.async_copy(
                    tab_hbm.at[:, pl.ds(c0, CB)], sts[b], isms[b]).wait()
                transpose_block(sts[b], obs[b], CB)
                o0 = pl.multiple_of(kb * (CB // 2), 8)
                pltpu.async_copy(
                    obs[b], out_hbm.at[pl.ds(o0, CB // 2)], osms[b])
        return carry

    lax.fori_loop(0, (NBLK_LO + 1 + NBUF - 1) // NBUF, bbody, 0)
    for b in range(NBUF):
        pltpu.make_async_copy(
            obs[b], out_hbm.at[pl.ds(0, CB // 2)], osms[b]).wait()

    @pl.when(wid == N_WORKERS - 1)
    def _tail():
        pltpu.async_copy(
            fix_hbm, ob0.at[pl.ds(0, 32)], isms[0]).wait()
        pltpu.async_copy(
            ob0.at[pl.ds(0, 32)],
            out_hbm.at[pl.ds(NBLK * CB // 2, 32)], osms[0]).wait()


_transpose_call = functools.partial(
    pl.kernel,
    mesh=plsc.VectorSubcoreMesh(core_axis_name="c", subcore_axis_name="s"),
    out_type=jax.ShapeDtypeStruct((NUM_EMB_PAD // 2, 128), jnp.float32),
    scratch_types=[
        pltpu.VMEM((EMBED_DIM, CB), jnp.float32),
        pltpu.VMEM((EMBED_DIM, CB), jnp.float32),
        pltpu.VMEM((CB // 2, 128), jnp.float32),
        pltpu.VMEM((CB // 2, 128), jnp.float32),
        pltpu.SemaphoreType.DMA,
        pltpu.SemaphoreType.DMA,
        pltpu.SemaphoreType.DMA,
        pltpu.SemaphoreType.DMA,
    ],
    compiler_params=pltpu.CompilerParams(needs_layout_passes=False),
)(_sc_transpose)


def _sc_gather(table_hbm, idx_hbm, out_hbm, idx_v, rows0, rows1,
               gsem0, gsem1, osem0, osem1):
    wid = lax.axis_index("s") * 2 + lax.axis_index("c")
    base = wid * PER_W

    rows = [rows0, rows1]
    gsems = [gsem0, gsem1]
    osems = [osem0, osem1]

    pltpu.sync_copy(idx_hbm.at[pl.ds(wid * N_CHUNK, N_CHUNK)], idx_v)

    gcp = [None] * N_CHUNK
    ocp = [None] * N_CHUNK
    for j in range(N_CHUNK + 1):
        if j < N_CHUNK:
            b = j % NBUF
            if j >= NBUF:
                ocp[j - NBUF].wait()
            gcp[j] = pltpu.async_copy(
                table_hbm.at[idx_v.at[j]], rows[b], gsems[b])
        if j >= 1:
            k = j - 1
            b = k % NBUF
            gcp[k].wait()
            ocp[k] = pltpu.async_copy(
                rows[b], out_hbm.at[pl.ds(base + k * CHUNK, CHUNK)], osems[b])
    for k in range(N_CHUNK - NBUF, N_CHUNK):
        ocp[k].wait()


_gather_call = functools.partial(
    pl.kernel,
    mesh=plsc.VectorSubcoreMesh(core_axis_name="c", subcore_axis_name="s"),
    out_type=jax.ShapeDtypeStruct((N_TOK, EMBED_DIM), jnp.float32),
    scratch_types=[
        pltpu.VMEM((N_CHUNK, CHUNK), jnp.int32),
        pltpu.VMEM((CHUNK, EMBED_DIM), jnp.float32),
        pltpu.VMEM((CHUNK, EMBED_DIM), jnp.float32),
        pltpu.SemaphoreType.DMA,
        pltpu.SemaphoreType.DMA,
        pltpu.SemaphoreType.DMA,
        pltpu.SemaphoreType.DMA,
    ],
    compiler_params=pltpu.CompilerParams(use_tc_tiling_on_sc=False),
)(_sc_gather)


def _mask_body(tokens_ref, pad_ref, seq_ref):
    pad_ref[...] = tokens_ref[...] == 0
    row = lax.broadcasted_iota(jnp.int32, (SEQ_LEN, SEQ_LEN), 0)
    col = lax.broadcasted_iota(jnp.int32, (SEQ_LEN, SEQ_LEN), 1)
    seq_ref[...] = col > row


_mask_call = pl.pallas_call(
    _mask_body,
    out_shape=(
        jax.ShapeDtypeStruct((BATCH, SEQ_LEN), jnp.bool_),
        jax.ShapeDtypeStruct((SEQ_LEN, SEQ_LEN), jnp.bool_),
    ),
)


@jax.jit
def kernel(tokens, table):
    tokens = tokens.astype(jnp.int32)
    idx = tokens.reshape(N_WORKERS * N_CHUNK, CHUNK)
    fix = table[NBLK * CB:].reshape(32, 128)
    tlin2 = _transpose_call(table.T, fix)
    tlin = tlin2.reshape(NUM_EMB_PAD, EMBED_DIM)
    features = _gather_call(tlin, idx).reshape(BATCH, SEQ_LEN, EMBED_DIM)
    pad, seq = _mask_call(tokens)
    return (features, (pad[:, None, :], seq))


# speedup vs baseline: 2.3906x; 2.3906x over previous
import functools

import jax
import jax.numpy as jnp
from jax import lax
from jax.experimental import pallas as pl
from jax.experimental.pallas import tpu as pltpu
from jax.experimental.pallas import tpu_sc as plsc

EMBED_DIM = 64
BATCH = 1024
SEQ_LEN = 200
N_TOK = BATCH * SEQ_LEN
N_WORKERS = 32
PER_W = N_TOK // N_WORKERS
CHUNK = 800
N_CHUNK = PER_W // CHUNK


NBUF = 2


def _sc_gather(table_hbm, idx_hbm, out_hbm, idx_v,
               rows0, rows1, gsem0, gsem1, osem0, osem1):
    wid = lax.axis_index("s") * 2 + lax.axis_index("c")
    base = wid * PER_W

    rows = [rows0, rows1]
    gsems = [gsem0, gsem1]
    osems = [osem0, osem1]

    pltpu.sync_copy(idx_hbm.at[pl.ds(wid * N_CHUNK, N_CHUNK)], idx_v)

    gcp = [None] * N_CHUNK
    ocp = [None] * N_CHUNK
    for j in range(N_CHUNK + 1):
        if j < N_CHUNK:
            b = j % NBUF
            if j >= NBUF:
                ocp[j - NBUF].wait()
            gcp[j] = pltpu.async_copy(
                table_hbm.at[idx_v.at[j]], rows[b], gsems[b])
        if j >= 1:
            k = j - 1
            b = k % NBUF
            gcp[k].wait()
            ocp[k] = pltpu.async_copy(
                rows[b], out_hbm.at[pl.ds(base + k * CHUNK, CHUNK)], osems[b])
    for k in range(N_CHUNK - NBUF, N_CHUNK):
        ocp[k].wait()


_gather_call = functools.partial(
    pl.kernel,
    mesh=plsc.VectorSubcoreMesh(core_axis_name="c", subcore_axis_name="s"),
    out_type=jax.ShapeDtypeStruct((N_TOK, EMBED_DIM), jnp.float32),
    scratch_types=[
        pltpu.VMEM((N_CHUNK, CHUNK), jnp.int32),
        pltpu.VMEM((CHUNK, EMBED_DIM), jnp.float32),
        pltpu.VMEM((CHUNK, EMBED_DIM), jnp.float32),
        pltpu.SemaphoreType.DMA,
        pltpu.SemaphoreType.DMA,
        pltpu.SemaphoreType.DMA,
        pltpu.SemaphoreType.DMA,
    ],
    compiler_params=pltpu.CompilerParams(use_tc_tiling_on_sc=False),
)(_sc_gather)


def _mask_body(tokens_ref, pad_ref, seq_ref):
    pad_ref[...] = tokens_ref[...] == 0
    row = lax.broadcasted_iota(jnp.int32, (SEQ_LEN, SEQ_LEN), 0)
    col = lax.broadcasted_iota(jnp.int32, (SEQ_LEN, SEQ_LEN), 1)
    seq_ref[...] = col > row


_mask_call = pl.pallas_call(
    _mask_body,
    out_shape=(
        jax.ShapeDtypeStruct((BATCH, SEQ_LEN), jnp.bool_),
        jax.ShapeDtypeStruct((SEQ_LEN, SEQ_LEN), jnp.bool_),
    ),
)


@jax.jit
def kernel(tokens, table):
    tokens = tokens.astype(jnp.int32)
    idx = tokens.reshape(N_WORKERS * N_CHUNK, CHUNK)
    features = _gather_call(table, idx).reshape(BATCH, SEQ_LEN, EMBED_DIM)
    pad, seq = _mask_call(tokens)
    return (features, (pad[:, None, :], seq))
